# baseline (speedup 1.0000x reference)
import functools

import jax
import jax.numpy as jnp
from jax import lax
from jax.experimental import pallas as pl
from jax.experimental.pallas import tpu as pltpu
from jax.experimental.pallas import tpu_sc as plsc

TOTAL_TOK = 16384
EMB = 128

_info = plsc.get_sparse_core_info()
_NC, _NS = _info.num_cores, _info.num_subcores
_NW = _NC * _NS
_B_PER_W = TOTAL_TOK // _NW
_CH = 128
_NCH = _B_PER_W // _CH
_NBUF = 2


def _gather_body(token_hbm, table_hbm, out_hbm, idx_v, rows0, rows1,
                 si0, si1, so0, so1):
    wid = lax.axis_index("s") * _NC + lax.axis_index("c")
    base = wid * _B_PER_W
    pltpu.sync_copy(token_hbm.at[pl.ds(base, _B_PER_W)], idx_v)

    rows = (rows0, rows1)
    si = (si0, si1)
    so = (so0, so1)

    def gather(k, buf):
        return pltpu.async_copy(
            table_hbm.at[idx_v.at[pl.ds(k * _CH, _CH)]], rows[buf], si[buf])

    gathers = [None] * _NCH
    outs = [None] * _NCH
    gathers[0] = gather(0, 0)
    for k in range(_NCH):
        b = k % _NBUF
        if k + 1 < _NCH:
            if k - 1 >= 0:
                outs[k - 1].wait()
            gathers[k + 1] = gather(k + 1, (k + 1) % _NBUF)
        gathers[k].wait()
        outs[k] = pltpu.async_copy(
            rows[b], out_hbm.at[pl.ds(base + k * _CH, _CH)], so[b])
    for k in range(max(0, _NCH - _NBUF), _NCH):
        outs[k].wait()


_mesh = plsc.VectorSubcoreMesh(core_axis_name="c", subcore_axis_name="s")

_gather = functools.partial(
    pl.kernel,
    mesh=_mesh,
    out_type=jax.ShapeDtypeStruct((TOTAL_TOK, EMB), jnp.float32),
    scratch_types=[
        pltpu.VMEM((_B_PER_W,), jnp.int32),
        pltpu.VMEM((_CH, EMB), jnp.float32),
        pltpu.VMEM((_CH, EMB), jnp.float32),
        pltpu.SemaphoreType.DMA,
        pltpu.SemaphoreType.DMA,
        pltpu.SemaphoreType.DMA,
        pltpu.SemaphoreType.DMA,
    ],
)(_gather_body)


@jax.jit
def kernel(token_ids, cu_seqlens, table):
    all_embs = _gather(token_ids.astype(jnp.int32), table)
    return (all_embs, cu_seqlens)


# speedup vs baseline: 1.0492x; 1.0492x over previous
import functools

import jax
import jax.numpy as jnp
from jax import lax
from jax.experimental import pallas as pl
from jax.experimental.pallas import tpu as pltpu
from jax.experimental.pallas import tpu_sc as plsc

TOTAL_TOK = 16384
EMB = 128

_info = plsc.get_sparse_core_info()
_NC, _NS = _info.num_cores, _info.num_subcores
_NW = _NC * _NS
_B_PER_W = TOTAL_TOK // _NW
_CH = 128
_NCH = _B_PER_W // _CH
_NBUF = 2


def _gather_body(token_hbm, table_hbm, out_hbm, idx_v, rows_v, sem):
    wid = lax.axis_index("s") * _NC + lax.axis_index("c")
    base = wid * _B_PER_W
    pltpu.sync_copy(token_hbm.at[pl.ds(base, _B_PER_W)], idx_v)
    pltpu.async_copy(table_hbm.at[idx_v], rows_v, sem).wait()
    pltpu.sync_copy(rows_v, out_hbm.at[pl.ds(base, _B_PER_W)])


_mesh = plsc.VectorSubcoreMesh(core_axis_name="c", subcore_axis_name="s")

_gather = functools.partial(
    pl.kernel,
    mesh=_mesh,
    out_type=jax.ShapeDtypeStruct((TOTAL_TOK, EMB), jnp.float32),
    scratch_types=[
        pltpu.VMEM((_B_PER_W,), jnp.int32),
        pltpu.VMEM((_B_PER_W, EMB), jnp.float32),
        pltpu.SemaphoreType.DMA,
    ],
)(_gather_body)


@jax.jit
def kernel(token_ids, cu_seqlens, table):
    all_embs = _gather(token_ids.astype(jnp.int32), table)
    return (all_embs, cu_seqlens)
